# Initial kernel scaffold; baseline (speedup 1.0000x reference)
import jax
import jax.numpy as jnp
from jax.experimental import pallas as pl


def kernel(x, edge_index, batch, W1, b1, W2, b2, Wc, bc):
    raise NotImplementedError("write your pallas kernel here")


# speedup vs baseline: 20.4837x; 20.4837x over previous
import functools

import jax
import jax.numpy as jnp
from jax import lax
from jax.experimental import pallas as pl
from jax.experimental.pallas import tpu as pltpu
from jax.experimental.pallas import tpu_sc as plsc

NN = 10000
EE = 320000
DD = 128
GG = 512

NC = 2
NS = 16
NW = NC * NS

NP = 10240
EP = 327680
EPW = EP // NW
ECH = 128
NCH = EPW // ECH
RPT = NP // NS

PPW = NP // NW
PCH = 80
PNCH = PPW // PCH
GPT = GG // NS

_mesh = plsc.VectorSubcoreMesh(core_axis_name="c", subcore_axis_name="s")


@functools.partial(
    pl.kernel,
    out_type=jax.ShapeDtypeStruct((NC, NP, 8), jnp.float32),
    mesh=_mesh,
    scratch_types=[
        pltpu.VMEM((NCH, ECH), jnp.int32),
        pltpu.VMEM((ECH, 8), jnp.float32),
        pltpu.VMEM_SHARED((NP, 8), jnp.float32),
    ],
)
def _sc_degree(dst_hbm, ones_hbm, zeros_hbm, out_hbm, idx_d, ones_v, acc):
    c = lax.axis_index("c")
    s = lax.axis_index("s")
    wid = s * NC + c
    pltpu.sync_copy(dst_hbm.at[wid], idx_d)
    pltpu.sync_copy(ones_hbm, ones_v)
    pltpu.sync_copy(zeros_hbm.at[pl.ds(s * RPT, RPT)], acc.at[pl.ds(s * RPT, RPT)])
    plsc.subcore_barrier()

    def body(j, _):
        pltpu.sync_copy(ones_v, acc.at[idx_d.at[j]], add=True)
        return 0

    lax.fori_loop(0, NCH, body, 0)
    plsc.subcore_barrier()
    pltpu.sync_copy(acc.at[pl.ds(s * RPT, RPT)], out_hbm.at[c, pl.ds(s * RPT, RPT)])


@functools.partial(
    pl.kernel,
    out_type=jax.ShapeDtypeStruct((NC, NP, DD), jnp.float32),
    mesh=_mesh,
    scratch_types=[
        pltpu.VMEM((NCH, ECH), jnp.int32),
        pltpu.VMEM((NCH, ECH), jnp.int32),
        pltpu.VMEM((ECH, DD), jnp.float32),
        pltpu.SemaphoreType.DMA,
        pltpu.VMEM_SHARED((NP, DD), jnp.float32),
    ],
)
def _sc_edge_scatter(table_hbm, src_hbm, dst_hbm, zeros_hbm, out_hbm,
                     idx_s, idx_d, rows, sem, acc):
    c = lax.axis_index("c")
    s = lax.axis_index("s")
    wid = s * NC + c
    pltpu.sync_copy(src_hbm.at[wid], idx_s)
    pltpu.sync_copy(dst_hbm.at[wid], idx_d)
    pltpu.sync_copy(zeros_hbm.at[pl.ds(s * RPT, RPT)], acc.at[pl.ds(s * RPT, RPT)])
    plsc.subcore_barrier()

    def body(j, _):
        pltpu.async_copy(table_hbm.at[idx_s.at[j]], rows, sem).wait()
        pltpu.sync_copy(rows, acc.at[idx_d.at[j]], add=True)
        return 0

    lax.fori_loop(0, NCH, body, 0)
    plsc.subcore_barrier()
    pltpu.sync_copy(acc.at[pl.ds(s * RPT, RPT)], out_hbm.at[c, pl.ds(s * RPT, RPT)])


@functools.partial(
    pl.kernel,
    out_type=jax.ShapeDtypeStruct((NC, GG, 8), jnp.float32),
    mesh=_mesh,
    scratch_types=[
        pltpu.VMEM((PNCH, PCH), jnp.int32),
        pltpu.VMEM((PPW, 8), jnp.float32),
        pltpu.VMEM_SHARED((GG, 8), jnp.float32),
    ],
)
def _sc_pool(p_hbm, batch_hbm, zeros_hbm, out_hbm, idx_b, rows, acc):
    c = lax.axis_index("c")
    s = lax.axis_index("s")
    wid = s * NC + c
    pltpu.sync_copy(batch_hbm.at[wid], idx_b)
    pltpu.sync_copy(p_hbm.at[wid], rows)
    pltpu.sync_copy(zeros_hbm.at[pl.ds(s * GPT, GPT)], acc.at[pl.ds(s * GPT, GPT)])
    plsc.subcore_barrier()

    def body(j, _):
        pltpu.sync_copy(rows.at[pl.ds(j * PCH, PCH)], acc.at[idx_b.at[j]], add=True)
        return 0

    lax.fori_loop(0, PNCH, body, 0)
    plsc.subcore_barrier()
    pltpu.sync_copy(acc.at[pl.ds(s * GPT, GPT)], out_hbm.at[c, pl.ds(s * GPT, GPT)])


_BN = 1024


def _dinv_blk(d0_ref, d1_ref):
    deg = d0_ref[:, :1] + d1_ref[:, :1] + 1.0
    return lax.rsqrt(deg)


def _real_row_mask(i, shape):
    gid = i * _BN + lax.broadcasted_iota(jnp.int32, shape, 0)
    return (gid < NN).astype(jnp.float32)


def _tc_scale_mm(d0_ref, d1_ref, x_ref, w_ref, o_ref):
    dinv = _dinv_blk(d0_ref, d1_ref)
    h = jnp.dot(x_ref[...], w_ref[...], preferred_element_type=jnp.float32)
    o_ref[...] = h * dinv


def _tc_combine_mm(d0_ref, d1_ref, s0_ref, s1_ref, hp_ref, b_ref, w_ref, o_ref):
    dinv = _dinv_blk(d0_ref, d1_ref)
    h = (s0_ref[...] + s1_ref[...] + hp_ref[...]) * dinv + b_ref[...]
    h = jnp.maximum(h, 0.0)
    o = jnp.dot(h, w_ref[...], preferred_element_type=jnp.float32) * dinv
    o_ref[...] = o * _real_row_mask(pl.program_id(0), o.shape)


def _tc_combine_cls(d0_ref, d1_ref, s0_ref, s1_ref, hp_ref, b_ref, w_ref, o_ref):
    dinv = _dinv_blk(d0_ref, d1_ref)
    h = (s0_ref[...] + s1_ref[...] + hp_ref[...]) * dinv + b_ref[...]
    p = jnp.dot(h, w_ref[...], preferred_element_type=jnp.float32)
    ones_col = (lax.broadcasted_iota(jnp.int32, p.shape, 1) == 2).astype(jnp.float32)
    o_ref[...] = (p + ones_col) * _real_row_mask(pl.program_id(0), p.shape)


def _tc_finish(p0_ref, p1_ref, bc_ref, o_ref):
    sums = p0_ref[...] + p1_ref[...]
    cnt = jnp.maximum(sums[:, 2:3], 1.0)
    o_ref[...] = sums / cnt + bc_ref[...]


def _nblk(i):
    return (i, 0)


def _rep(i):
    return (0, 0)


_D8 = pl.BlockSpec((_BN, 8), _nblk)
_DN = pl.BlockSpec((_BN, DD), _nblk)
_WW = pl.BlockSpec((DD, DD), _rep)
_BB = pl.BlockSpec((1, DD), _rep)
_GRID = NP // _BN


def kernel(x, edge_index, batch, W1, b1, W2, b2, Wc, bc):
    pad_idx = NN + (jnp.arange(EP - EE, dtype=jnp.int32) % (NP - NN))
    src3 = jnp.concatenate([edge_index[0], pad_idx]).reshape(NW, NCH, ECH)
    dst3 = jnp.concatenate([edge_index[1], pad_idx]).reshape(NW, NCH, ECH)
    xp = jnp.pad(x, ((0, NP - NN), (0, 0)))
    zeros_nd = jnp.zeros((NP, DD), jnp.float32)
    zeros_n8 = jnp.zeros((NP, 8), jnp.float32)
    ones_e8 = jnp.ones((ECH, 8), jnp.float32)

    deg = _sc_degree(dst3, ones_e8, zeros_n8)
    d0, d1 = deg[0], deg[1]

    hp1 = pl.pallas_call(
        _tc_scale_mm,
        grid=(_GRID,),
        in_specs=[_D8, _D8, _DN, _WW],
        out_specs=_DN,
        out_shape=jax.ShapeDtypeStruct((NP, DD), jnp.float32),
    )(d0, d1, xp, W1)

    s1 = _sc_edge_scatter(hp1, src3, dst3, zeros_nd)

    hp2 = pl.pallas_call(
        _tc_combine_mm,
        grid=(_GRID,),
        in_specs=[_D8, _D8, _DN, _DN, _DN, _BB, _WW],
        out_specs=_DN,
        out_shape=jax.ShapeDtypeStruct((NP, DD), jnp.float32),
    )(d0, d1, s1[0], s1[1], hp1, b1.reshape(1, DD), W2)

    s2 = _sc_edge_scatter(hp2, src3, dst3, zeros_nd)

    wc8 = jnp.pad(Wc, ((0, 0), (0, 8 - Wc.shape[1])))
    p8 = pl.pallas_call(
        _tc_combine_cls,
        grid=(_GRID,),
        in_specs=[_D8, _D8, _DN, _DN, _DN, _BB,
                  pl.BlockSpec((DD, 8), _rep)],
        out_specs=_D8,
        out_shape=jax.ShapeDtypeStruct((NP, 8), jnp.float32),
    )(d0, d1, s2[0], s2[1], hp2, b2.reshape(1, DD), wc8)

    p8r = p8.reshape(NW, PPW, 8)
    bpad = jnp.pad(batch, (0, NP - NN)).reshape(NW, PNCH, PCH)
    pool = _sc_pool(p8r, bpad, zeros_n8)

    out8 = pl.pallas_call(
        _tc_finish,
        grid=(1,),
        in_specs=[pl.BlockSpec((GG, 8), _rep), pl.BlockSpec((GG, 8), _rep),
                  pl.BlockSpec((1, 8), _rep)],
        out_specs=pl.BlockSpec((GG, 8), _rep),
        out_shape=jax.ShapeDtypeStruct((GG, 8), jnp.float32),
    )(pool[0], pool[1], jnp.pad(bc, (0, 6)).reshape(1, 8))

    return out8[:, :2]


# speedup vs baseline: 25.3044x; 1.2353x over previous
import functools

import jax
import jax.numpy as jnp
from jax import lax
from jax.experimental import pallas as pl
from jax.experimental.pallas import tpu as pltpu
from jax.experimental.pallas import tpu_sc as plsc

NN = 10000
EE = 320000
DD = 128
GG = 512

NC = 2
NS = 16
NW = NC * NS

NP = 10240
EP = 327680
EPW = EP // NW
ECH = 128
NCH = EPW // ECH
BCH = 8
NB = NCH // BCH
RPT = NP // NS

PPW = NP // NW
PCH = 80
PNCH = PPW // PCH
GPT = GG // NS

_mesh = plsc.VectorSubcoreMesh(core_axis_name="c", subcore_axis_name="s")


@functools.partial(
    pl.kernel,
    out_type=jax.ShapeDtypeStruct((NC, NP, 8), jnp.float32),
    mesh=_mesh,
    scratch_types=[
        pltpu.VMEM((NCH, 2, ECH), jnp.int32),
        pltpu.VMEM((ECH, 8), jnp.float32),
        pltpu.VMEM_SHARED((NP, 8), jnp.float32),
    ],
)
def _sc_degree(pairs_hbm, ones_hbm, zeros_hbm, out_hbm, idx_v, ones_v, acc):
    c = lax.axis_index("c")
    s = lax.axis_index("s")
    wid = s * NC + c
    pltpu.sync_copy(pairs_hbm.at[wid], idx_v)
    pltpu.sync_copy(ones_hbm, ones_v)
    pltpu.sync_copy(zeros_hbm.at[pl.ds(s * RPT, RPT)], acc.at[pl.ds(s * RPT, RPT)])
    plsc.subcore_barrier()

    def body(j, _):
        pltpu.sync_copy(ones_v, acc.at[idx_v.at[j, 1]], add=True)
        return 0

    lax.fori_loop(0, NCH, body, 0)
    plsc.subcore_barrier()
    pltpu.sync_copy(acc.at[pl.ds(s * RPT, RPT)], out_hbm.at[c, pl.ds(s * RPT, RPT)])


@functools.partial(
    pl.kernel,
    out_type=jax.ShapeDtypeStruct((NC, NP, DD), jnp.float32),
    mesh=_mesh,
    scratch_types=[
        pltpu.VMEM((BCH, 2, ECH), jnp.int32),
        pltpu.VMEM((BCH, 2, ECH), jnp.int32),
        pltpu.VMEM((ECH, DD), jnp.float32),
        pltpu.VMEM((ECH, DD), jnp.float32),
        pltpu.SemaphoreType.DMA,
        pltpu.SemaphoreType.DMA,
        pltpu.SemaphoreType.DMA,
        pltpu.SemaphoreType.DMA,
        pltpu.VMEM_SHARED((NP, DD), jnp.float32),
    ],
)
def _sc_edge_scatter(table_hbm, pairs_hbm, zeros_hbm, out_hbm,
                     set0, set1, rows0, rows1, sem0, sem1, semi0, semi1, acc):
    c = lax.axis_index("c")
    s = lax.axis_index("s")
    wid = s * NC + c
    rows = (rows0, rows1)
    sems = (sem0, sem1)

    def stage(b, dst_set, sem):
        return pltpu.async_copy(pairs_hbm.at[wid, pl.ds(b * BCH, BCH)], dst_set, sem)

    def stage_wait(b, dst_set, sem):
        pltpu.make_async_copy(
            pairs_hbm.at[wid, pl.ds(b * BCH, BCH)], dst_set, sem).wait()

    def block(b, cur, nxt, sem_nxt, stage_next, prefetch_next):
        if stage_next:
            stage(b + 1, nxt, sem_nxt)
        for i in range(BCH):
            rb, ro = rows[i % 2], rows[(i + 1) % 2]
            sb, so = sems[i % 2], sems[(i + 1) % 2]
            pltpu.make_async_copy(table_hbm.at[cur.at[i, 0]], rb, sb).wait()
            if i + 1 < BCH:
                pltpu.async_copy(table_hbm.at[cur.at[i + 1, 0]], ro, so)
            elif prefetch_next:
                stage_wait(b + 1, nxt, sem_nxt)
                pltpu.async_copy(table_hbm.at[nxt.at[0, 0]], ro, so)
            pltpu.sync_copy(rb, acc.at[cur.at[i, 1]], add=True)

    stage(0, set0, semi0)
    stage_wait(0, set0, semi0)
    pltpu.async_copy(table_hbm.at[set0.at[0, 0]], rows0, sem0)
    pltpu.sync_copy(zeros_hbm.at[pl.ds(s * RPT, RPT)], acc.at[pl.ds(s * RPT, RPT)])
    plsc.subcore_barrier()

    def body(t, _):
        b = 2 * t
        block(b, set0, set1, semi1, True, True)
        block(b + 1, set1, set0, semi0, True, True)
        return 0

    lax.fori_loop(0, NB // 2 - 1, body, 0)
    block(NB - 2, set0, set1, semi1, True, True)
    block(NB - 1, set1, set0, semi0, False, False)
    plsc.subcore_barrier()
    pltpu.sync_copy(acc.at[pl.ds(s * RPT, RPT)], out_hbm.at[c, pl.ds(s * RPT, RPT)])


@functools.partial(
    pl.kernel,
    out_type=jax.ShapeDtypeStruct((NC, GG, 8), jnp.float32),
    mesh=_mesh,
    scratch_types=[
        pltpu.VMEM((PNCH, PCH), jnp.int32),
        pltpu.VMEM((PPW, 8), jnp.float32),
        pltpu.VMEM_SHARED((GG, 8), jnp.float32),
    ],
)
def _sc_pool(p_hbm, batch_hbm, zeros_hbm, out_hbm, idx_b, rows, acc):
    c = lax.axis_index("c")
    s = lax.axis_index("s")
    wid = s * NC + c
    pltpu.sync_copy(batch_hbm.at[wid], idx_b)
    pltpu.sync_copy(p_hbm.at[wid], rows)
    pltpu.sync_copy(zeros_hbm.at[pl.ds(s * GPT, GPT)], acc.at[pl.ds(s * GPT, GPT)])
    plsc.subcore_barrier()

    def body(j, _):
        pltpu.sync_copy(rows.at[pl.ds(j * PCH, PCH)], acc.at[idx_b.at[j]], add=True)
        return 0

    lax.fori_loop(0, PNCH, body, 0)
    plsc.subcore_barrier()
    pltpu.sync_copy(acc.at[pl.ds(s * GPT, GPT)], out_hbm.at[c, pl.ds(s * GPT, GPT)])


_BN = 1024


def _dinv_blk(d0_ref, d1_ref):
    deg = d0_ref[:, :1] + d1_ref[:, :1] + 1.0
    return lax.rsqrt(deg)


def _real_row_mask(i, shape):
    gid = i * _BN + lax.broadcasted_iota(jnp.int32, shape, 0)
    return (gid < NN).astype(jnp.float32)


def _tc_scale_mm(d0_ref, d1_ref, x_ref, w_ref, o_ref):
    dinv = _dinv_blk(d0_ref, d1_ref)
    h = jnp.dot(x_ref[...], w_ref[...], preferred_element_type=jnp.float32)
    o_ref[...] = h * dinv


def _tc_combine_mm(d0_ref, d1_ref, s0_ref, s1_ref, hp_ref, b_ref, w_ref, o_ref):
    dinv = _dinv_blk(d0_ref, d1_ref)
    h = (s0_ref[...] + s1_ref[...] + hp_ref[...]) * dinv + b_ref[...]
    h = jnp.maximum(h, 0.0)
    o = jnp.dot(h, w_ref[...], preferred_element_type=jnp.float32) * dinv
    o_ref[...] = o * _real_row_mask(pl.program_id(0), o.shape)


def _tc_combine_cls(d0_ref, d1_ref, s0_ref, s1_ref, hp_ref, b_ref, w_ref, o_ref):
    dinv = _dinv_blk(d0_ref, d1_ref)
    h = (s0_ref[...] + s1_ref[...] + hp_ref[...]) * dinv + b_ref[...]
    p = jnp.dot(h, w_ref[...], preferred_element_type=jnp.float32)
    ones_col = (lax.broadcasted_iota(jnp.int32, p.shape, 1) == 2).astype(jnp.float32)
    o_ref[...] = (p + ones_col) * _real_row_mask(pl.program_id(0), p.shape)


def _tc_finish(p0_ref, p1_ref, bc_ref, o_ref):
    sums = p0_ref[...] + p1_ref[...]
    cnt = jnp.maximum(sums[:, 2:3], 1.0)
    o_ref[...] = sums / cnt + bc_ref[...]


def _nblk(i):
    return (i, 0)


def _rep(i):
    return (0, 0)


_D8 = pl.BlockSpec((_BN, 8), _nblk)
_DN = pl.BlockSpec((_BN, DD), _nblk)
_WW = pl.BlockSpec((DD, DD), _rep)
_BB = pl.BlockSpec((1, DD), _rep)
_GRID = NP // _BN


def kernel(x, edge_index, batch, W1, b1, W2, b2, Wc, bc):
    pad_idx = NN + (jnp.arange(EP - EE, dtype=jnp.int32) % (NP - NN))
    src3 = jnp.concatenate([edge_index[0], pad_idx]).reshape(NW, NCH, 1, ECH)
    dst3 = jnp.concatenate([edge_index[1], pad_idx]).reshape(NW, NCH, 1, ECH)
    pairs = jnp.concatenate([src3, dst3], axis=2)
    xp = jnp.pad(x, ((0, NP - NN), (0, 0)))
    zeros_nd = jnp.zeros((NP, DD), jnp.float32)
    zeros_n8 = jnp.zeros((NP, 8), jnp.float32)
    ones_e8 = jnp.ones((ECH, 8), jnp.float32)

    deg = _sc_degree(pairs, ones_e8, zeros_n8)
    d0, d1 = deg[0], deg[1]

    hp1 = pl.pallas_call(
        _tc_scale_mm,
        grid=(_GRID,),
        in_specs=[_D8, _D8, _DN, _WW],
        out_specs=_DN,
        out_shape=jax.ShapeDtypeStruct((NP, DD), jnp.float32),
    )(d0, d1, xp, W1)

    s1 = _sc_edge_scatter(hp1, pairs, zeros_nd)

    hp2 = pl.pallas_call(
        _tc_combine_mm,
        grid=(_GRID,),
        in_specs=[_D8, _D8, _DN, _DN, _DN, _BB, _WW],
        out_specs=_DN,
        out_shape=jax.ShapeDtypeStruct((NP, DD), jnp.float32),
    )(d0, d1, s1[0], s1[1], hp1, b1.reshape(1, DD), W2)

    s2 = _sc_edge_scatter(hp2, pairs, zeros_nd)

    wc8 = jnp.pad(Wc, ((0, 0), (0, 8 - Wc.shape[1])))
    p8 = pl.pallas_call(
        _tc_combine_cls,
        grid=(_GRID,),
        in_specs=[_D8, _D8, _DN, _DN, _DN, _BB,
                  pl.BlockSpec((DD, 8), _rep)],
        out_specs=_D8,
        out_shape=jax.ShapeDtypeStruct((NP, 8), jnp.float32),
    )(d0, d1, s2[0], s2[1], hp2, b2.reshape(1, DD), wc8)

    p8r = p8.reshape(NW, PPW, 8)
    bpad = jnp.pad(batch, (0, NP - NN)).reshape(NW, PNCH, PCH)
    pool = _sc_pool(p8r, bpad, zeros_n8)

    out8 = pl.pallas_call(
        _tc_finish,
        grid=(1,),
        in_specs=[pl.BlockSpec((GG, 8), _rep), pl.BlockSpec((GG, 8), _rep),
                  pl.BlockSpec((1, 8), _rep)],
        out_specs=pl.BlockSpec((GG, 8), _rep),
        out_shape=jax.ShapeDtypeStruct((GG, 8), jnp.float32),
    )(pool[0], pool[1], jnp.pad(bc, (0, 6)).reshape(1, 8))

    return out8[:, :2]


# speedup vs baseline: 26.0397x; 1.0291x over previous
import functools

import jax
import jax.numpy as jnp
from jax import lax
from jax.experimental import pallas as pl
from jax.experimental.pallas import tpu as pltpu
from jax.experimental.pallas import tpu_sc as plsc

NN = 10000
EE = 320000
DD = 128
GG = 512

NC = 2
NS = 16
NW = NC * NS

NP = 10240
EP = 327680
EPW = EP // NW
ECH = 128
NCH = EPW // ECH
BCH = 8
NB = NCH // BCH
RPT = NP // NS

PPW = NP // NW
PCH = 80
PNCH = PPW // PCH
GPT = GG // NS

_mesh = plsc.VectorSubcoreMesh(core_axis_name="c", subcore_axis_name="s")


@functools.partial(
    pl.kernel,
    out_type=jax.ShapeDtypeStruct((NC, NP, 8), jnp.float32),
    mesh=_mesh,
    scratch_types=[
        pltpu.VMEM((NCH, 2, ECH), jnp.int32),
        pltpu.VMEM((ECH, 8), jnp.float32),
        pltpu.SemaphoreType.DMA,
        pltpu.VMEM_SHARED((NP, 8), jnp.float32),
    ],
)
def _sc_degree(pairs_hbm, ones_hbm, zeros_hbm, out_hbm, idx_v, ones_v, sem, acc):
    c = lax.axis_index("c")
    s = lax.axis_index("s")
    wid = s * NC + c
    pltpu.sync_copy(pairs_hbm.at[wid], idx_v)
    pltpu.sync_copy(ones_hbm, ones_v)
    pltpu.sync_copy(zeros_hbm.at[pl.ds(s * RPT, RPT)], acc.at[pl.ds(s * RPT, RPT)])
    plsc.subcore_barrier()

    def body(g, _):
        for u in range(8):
            pltpu.async_copy(ones_v, acc.at[idx_v.at[g * 8 + u, 1]], sem, add=True)
        for u in range(8):
            pltpu.make_async_copy(ones_v, acc.at[idx_v.at[g * 8 + u, 1]], sem).wait()
        return 0

    lax.fori_loop(0, NCH // 8, body, 0)
    plsc.subcore_barrier()
    pltpu.sync_copy(acc.at[pl.ds(s * RPT, RPT)], out_hbm.at[c, pl.ds(s * RPT, RPT)])


@functools.partial(
    pl.kernel,
    out_type=jax.ShapeDtypeStruct((NC, NP, DD), jnp.float32),
    mesh=_mesh,
    scratch_types=[
        pltpu.VMEM((BCH, 2, ECH), jnp.int32),
        pltpu.VMEM((BCH, 2, ECH), jnp.int32),
        pltpu.VMEM((ECH, DD), jnp.float32),
        pltpu.VMEM((ECH, DD), jnp.float32),
        pltpu.SemaphoreType.DMA,
        pltpu.SemaphoreType.DMA,
        pltpu.SemaphoreType.DMA,
        pltpu.SemaphoreType.DMA,
        pltpu.SemaphoreType.DMA,
        pltpu.SemaphoreType.DMA,
        pltpu.VMEM_SHARED((NP, DD), jnp.float32),
    ],
)
def _sc_edge_scatter(table_hbm, pairs_hbm, zeros_hbm, out_hbm,
                     set0, set1, rows0, rows1, semg0, semg1, semc0, semc1,
                     semi0, semi1, acc):
    c = lax.axis_index("c")
    s = lax.axis_index("s")
    wid = s * NC + c
    rows = (rows0, rows1)
    semg = (semg0, semg1)
    semc = (semc0, semc1)

    def stage(b, dst_set, sem):
        return pltpu.async_copy(pairs_hbm.at[wid, pl.ds(b * BCH, BCH)], dst_set, sem)

    def stage_wait(b, dst_set, sem):
        pltpu.make_async_copy(
            pairs_hbm.at[wid, pl.ds(b * BCH, BCH)], dst_set, sem).wait()

    def block(b, cur, nxt, sem_nxt, stage_next, prefetch_next, first=False):
        for i in range(BCH):
            rb, ro = rows[i % 2], rows[(i + 1) % 2]
            sgb, sgo = semg[i % 2], semg[(i + 1) % 2]
            scb, sco = semc[i % 2], semc[(i + 1) % 2]
            pltpu.make_async_copy(table_hbm.at[cur.at[i, 0]], rb, sgb).wait()
            if i == 0 and stage_next:
                stage(b + 1, nxt, sem_nxt)
            if i + 1 < BCH:
                pltpu.async_copy(table_hbm.at[cur.at[i + 1, 0]], ro, sgo)
            elif prefetch_next:
                stage_wait(b + 1, nxt, sem_nxt)
                pltpu.async_copy(table_hbm.at[nxt.at[0, 0]], ro, sgo)
            pltpu.sync_copy(rb, acc.at[cur.at[i, 1]], add=True)

    stage(0, set0, semi0)
    stage_wait(0, set0, semi0)
    pltpu.async_copy(table_hbm.at[set0.at[0, 0]], rows0, semg[0])
    pltpu.sync_copy(zeros_hbm.at[pl.ds(s * RPT, RPT)], acc.at[pl.ds(s * RPT, RPT)])
    plsc.subcore_barrier()

    block(0, set0, set1, semi1, True, True, first=True)

    def body(t, _):
        b = 2 * t + 1
        block(b, set1, set0, semi0, True, True)
        block(b + 1, set0, set1, semi1, True, True)
        return 0

    lax.fori_loop(0, (NB - 2) // 2, body, 0)
    block(NB - 1, set1, set0, semi0, False, False)
    plsc.subcore_barrier()
    pltpu.sync_copy(acc.at[pl.ds(s * RPT, RPT)], out_hbm.at[c, pl.ds(s * RPT, RPT)])


_BN = 1024


def _dinv_blk(d0_ref, d1_ref):
    deg = d0_ref[:, :1] + d1_ref[:, :1] + 1.0
    return lax.rsqrt(deg)


def _real_row_mask(i, shape):
    gid = i * _BN + lax.broadcasted_iota(jnp.int32, shape, 0)
    return (gid < NN).astype(jnp.float32)


def _tc_scale_mm(d0_ref, d1_ref, x_ref, w_ref, o_ref):
    dinv = _dinv_blk(d0_ref, d1_ref)
    h = jnp.dot(x_ref[...], w_ref[...], preferred_element_type=jnp.float32)
    o_ref[...] = h * dinv


def _tc_combine_mm(d0_ref, d1_ref, s0_ref, s1_ref, hp_ref, b_ref, w_ref, o_ref):
    dinv = _dinv_blk(d0_ref, d1_ref)
    h = (s0_ref[...] + s1_ref[...] + hp_ref[...]) * dinv + b_ref[...]
    h = jnp.maximum(h, 0.0)
    o = jnp.dot(h, w_ref[...], preferred_element_type=jnp.float32) * dinv
    o_ref[...] = o * _real_row_mask(pl.program_id(0), o.shape)


def _tc_cls_pool(d0_ref, d1_ref, s0_ref, s1_ref, hp_ref, b_ref, w_ref,
                 batch_ref, bc_ref, o_ref):
    i = pl.program_id(0)
    dinv = _dinv_blk(d0_ref, d1_ref)
    h = (s0_ref[...] + s1_ref[...] + hp_ref[...]) * dinv + b_ref[...]
    p = jnp.dot(h, w_ref[...], preferred_element_type=jnp.float32)
    ones_col = (lax.broadcasted_iota(jnp.int32, p.shape, 1) == 2).astype(jnp.float32)
    p = (p + ones_col) * _real_row_mask(i, p.shape)
    onehot = (batch_ref[...] ==
              lax.broadcasted_iota(jnp.int32, (_BN, GG), 1)).astype(jnp.float32)
    part = lax.dot_general(onehot, p, (((0,), (0,)), ((), ())),
                           preferred_element_type=jnp.float32)

    @pl.when(i == 0)
    def _():
        o_ref[...] = part

    @pl.when(i > 0)
    def _():
        o_ref[...] += part

    @pl.when(i == _GRID - 1)
    def _():
        sums = o_ref[...]
        cnt = jnp.maximum(sums[:, 2:3], 1.0)
        o_ref[...] = sums / cnt + bc_ref[...]


def _nblk(i):
    return (i, 0)


def _rep(i):
    return (0, 0)


_D8 = pl.BlockSpec((_BN, 8), _nblk)
_DN = pl.BlockSpec((_BN, DD), _nblk)
_WW = pl.BlockSpec((DD, DD), _rep)
_BB = pl.BlockSpec((1, DD), _rep)
_GRID = NP // _BN


def kernel(x, edge_index, batch, W1, b1, W2, b2, Wc, bc):
    pad_idx = NN + (jnp.arange(EP - EE, dtype=jnp.int32) % (NP - NN))
    src3 = jnp.concatenate([edge_index[0], pad_idx]).reshape(NW, NCH, 1, ECH)
    dst3 = jnp.concatenate([edge_index[1], pad_idx]).reshape(NW, NCH, 1, ECH)
    pairs = jnp.concatenate([src3, dst3], axis=2)
    xp = jnp.pad(x, ((0, NP - NN), (0, 0)))
    zeros_nd = jnp.zeros((NP, DD), jnp.float32)
    zeros_n8 = jnp.zeros((NP, 8), jnp.float32)
    ones_e8 = jnp.ones((ECH, 8), jnp.float32)

    deg = _sc_degree(pairs, ones_e8, zeros_n8)
    d0, d1 = deg[0], deg[1]

    hp1 = pl.pallas_call(
        _tc_scale_mm,
        grid=(_GRID,),
        in_specs=[_D8, _D8, _DN, _WW],
        out_specs=_DN,
        out_shape=jax.ShapeDtypeStruct((NP, DD), jnp.float32),
    )(d0, d1, xp, W1)

    s1 = _sc_edge_scatter(hp1, pairs, zeros_nd)

    hp2 = pl.pallas_call(
        _tc_combine_mm,
        grid=(_GRID,),
        in_specs=[_D8, _D8, _DN, _DN, _DN, _BB, _WW],
        out_specs=_DN,
        out_shape=jax.ShapeDtypeStruct((NP, DD), jnp.float32),
    )(d0, d1, s1[0], s1[1], hp1, b1.reshape(1, DD), W2)

    s2 = _sc_edge_scatter(hp2, pairs, zeros_nd)

    wc8 = jnp.pad(Wc, ((0, 0), (0, 8 - Wc.shape[1])))
    bpad = jnp.pad(batch, (0, NP - NN)).reshape(NP, 1)
    out8 = pl.pallas_call(
        _tc_cls_pool,
        grid=(_GRID,),
        in_specs=[_D8, _D8, _DN, _DN, _DN, _BB,
                  pl.BlockSpec((DD, 8), _rep),
                  pl.BlockSpec((_BN, 1), _nblk),
                  pl.BlockSpec((1, 8), _rep)],
        out_specs=pl.BlockSpec((GG, 8), _rep),
        out_shape=jax.ShapeDtypeStruct((GG, 8), jnp.float32),
    )(d0, d1, s2[0], s2[1], hp2, b2.reshape(1, DD), wc8, bpad,
      jnp.pad(bc, (0, 6)).reshape(1, 8))

    return out8[:, :2]
